# baseline (speedup 1.0000x reference)
import functools

import jax
import jax.numpy as jnp
from jax import lax
from jax.experimental import pallas as pl
from jax.experimental.pallas import tpu as pltpu
from jax.experimental.pallas import tpu_sc as plsc

_LANES = 16


@functools.lru_cache(maxsize=None)
def _make_sc_gather(B, L, N):
    info = plsc.get_sparse_core_info()
    NC, NS = info.num_cores, info.num_subcores
    NW = NC * NS
    assert (B * L) % NW == 0
    slabs_per_tile = (B * L) // NW
    KN = N // _LANES
    Q = L // 2

    mesh = plsc.VectorSubcoreMesh(core_axis_name="c", subcore_axis_name="s")

    @functools.partial(
        pl.kernel,
        out_type=jax.ShapeDtypeStruct((B, L, Q, 2 * N), jnp.float32),
        mesh=mesh,
        scratch_types=[
            pltpu.VMEM((L, N), jnp.float32),
            pltpu.VMEM((Q, 2 * N), jnp.float32),
            pltpu.VMEM((Q, 2 * N), jnp.float32),
            pltpu.SemaphoreType.DMA,
            pltpu.SemaphoreType.DMA,
        ],
    )
    def run(inc_hbm, x_hbm, table_v, slab0, slab1, sem0, sem1):
        wid = lax.axis_index("s") * NC + lax.axis_index("c")
        tpb = L // slabs_per_tile
        b = wid // tpb
        r0 = (wid % tpb) * slabs_per_tile

        pltpu.sync_copy(inc_hbm.at[b], table_v)

        slabs = (slab0, slab1)
        sems = (sem0, sem1)

        def build(slab, r):
            s = lax.shift_right_logical(r, 1)
            d = lax.bitwise_and(r, 1)

            def q_body(q, carry):
                m0 = s + q
                m1 = m0 + d
                for k in range(KN):
                    slab[q, pl.ds(k * _LANES, _LANES)] = (
                        table_v[m0, pl.ds(k * _LANES, _LANES)]
                    )
                    slab[q, pl.ds(N + k * _LANES, _LANES)] = (
                        table_v[m1, pl.ds(k * _LANES, _LANES)]
                    )
                return carry

            lax.fori_loop(0, Q, q_body, 0)

        def outer(g, carry):
            for buf in range(2):
                r = r0 + g * 2 + buf

                @pl.when(g > 0)
                def _wait():
                    pltpu.make_async_copy(
                        slabs[buf], x_hbm.at[b, r0], sems[buf]
                    ).wait()

                build(slabs[buf], r)
                pltpu.async_copy(slabs[buf], x_hbm.at[b, r], sems[buf])
            return carry

        lax.fori_loop(0, slabs_per_tile // 2, outer, 0)

        for buf in range(2):
            pltpu.make_async_copy(
                slabs[buf], x_hbm.at[b, r0], sems[buf]
            ).wait()

    return run


@functools.lru_cache(maxsize=None)
def _make_full_sc(B, L, N, out_B=None):
    if out_B is None:
        out_B = B
    info = plsc.get_sparse_core_info()
    NC, NS = info.num_cores, info.num_subcores
    NW = NC * NS
    assert (B * L) % NW == 0
    slabs_per_tile = (B * L) // NW
    KN = N // _LANES

    mesh = plsc.VectorSubcoreMesh(core_axis_name="c", subcore_axis_name="s")

    @functools.partial(
        pl.kernel,
        out_type=jax.ShapeDtypeStruct((out_B, L, L, 3 * N), jnp.float32),
        mesh=mesh,
        scratch_types=[
            pltpu.VMEM((L, N), jnp.float32),
            pltpu.VMEM((L, 3 * N), jnp.float32),
            pltpu.VMEM((L, 3 * N), jnp.float32),
            pltpu.SemaphoreType.DMA,
            pltpu.SemaphoreType.DMA,
        ],
    )
    def run(inc_hbm, out_hbm, table_v, slab0, slab1, sem0, sem1):
        wid = lax.axis_index("s") * NC + lax.axis_index("c")
        tpb = L // slabs_per_tile
        b = wid // tpb
        r0 = (wid % tpb) * slabs_per_tile

        pltpu.sync_copy(inc_hbm.at[b], table_v)

        slabs = (slab0, slab1)
        sems = (sem0, sem1)

        def init_body(c, carry):
            for k in range(KN):
                v = table_v[c, pl.ds(k * _LANES, _LANES)]
                slab0[c, pl.ds(k * _LANES, _LANES)] = v
                slab1[c, pl.ds(k * _LANES, _LANES)] = v
            return carry

        lax.fori_loop(0, L, init_body, 0)

        def build(slab, r):
            row = [table_v[r, pl.ds(k * _LANES, _LANES)] for k in range(KN)]

            def c_body(c, carry):
                m = lax.shift_right_logical(r + c, 1)
                for k in range(KN):
                    slab[c, pl.ds(N + k * _LANES, _LANES)] = row[k]
                    slab[c, pl.ds(2 * N + k * _LANES, _LANES)] = (
                        table_v[m, pl.ds(k * _LANES, _LANES)]
                    )
                return carry

            lax.fori_loop(0, L, c_body, 0)

        def outer(g, carry):
            for buf in range(2):
                r = r0 + g * 2 + buf

                @pl.when(g > 0)
                def _wait():
                    pltpu.make_async_copy(
                        slabs[buf], out_hbm.at[b, r0], sems[buf]
                    ).wait()

                build(slabs[buf], r)
                pltpu.async_copy(slabs[buf], out_hbm.at[b, r], sems[buf])
            return carry

        lax.fori_loop(0, slabs_per_tile // 2, outer, 0)

        for buf in range(2):
            pltpu.make_async_copy(
                slabs[buf], out_hbm.at[b, r0], sems[buf]
            ).wait()

    return run


@functools.lru_cache(maxsize=None)
def _make_tc_assemble(B, L, N):
    def body(inc_ref, x_ref, o_ref):
        r = pl.program_id(1)
        t = inc_ref[0]
        sel = lax.broadcasted_iota(jnp.int32, (L, L), 1)
        s2 = (sel == r).astype(jnp.float32)
        p2 = jnp.dot(s2, t, preferred_element_type=jnp.float32)
        p3 = x_ref[0, 0].reshape(L, N)
        o_ref[0, 0] = jnp.concatenate([t, p2, p3], axis=1)

    return pl.pallas_call(
        body,
        grid=(B, L),
        in_specs=[
            pl.BlockSpec((1, L, N), lambda b, r: (b, 0, 0)),
            pl.BlockSpec((1, 1, L // 2, 2 * N), lambda b, r: (b, r, 0, 0)),
        ],
        out_specs=pl.BlockSpec((1, 1, L, 3 * N), lambda b, r: (b, r, 0, 0)),
        out_shape=jax.ShapeDtypeStruct((B, L, L, 3 * N), jnp.float32),
    )


def kernel(incoming):
    B, L, N = incoming.shape
    x = _make_sc_gather(B, L, N)(incoming)
    p3 = x.reshape(B, L, L, N)
    z = jnp.zeros((B, L, N), jnp.float32)
    a1 = jnp.concatenate([incoming, z, z], axis=-1)
    a2 = jnp.concatenate([z, incoming, z], axis=-1)
    out12 = a1[:, None, :, :] + a2[:, :, None, :]
    return lax.dynamic_update_slice(out12, p3, (0, 0, 0, 2 * N))


# speedup vs baseline: 1.0007x; 1.0007x over previous
import functools

import jax
import jax.numpy as jnp
from jax import lax
from jax.experimental import pallas as pl
from jax.experimental.pallas import tpu as pltpu
from jax.experimental.pallas import tpu_sc as plsc

_LANES = 16


@functools.lru_cache(maxsize=None)
def _make_sc_gather(B, L, N):
    info = plsc.get_sparse_core_info()
    NC, NS = info.num_cores, info.num_subcores
    NW = NC * NS
    assert (B * L) % NW == 0
    slabs_per_tile = (B * L) // NW
    KN = N // _LANES
    Q = L // 2

    mesh = plsc.VectorSubcoreMesh(core_axis_name="c", subcore_axis_name="s")

    @functools.partial(
        pl.kernel,
        out_type=jax.ShapeDtypeStruct((B, L, Q, 2 * N), jnp.float32),
        mesh=mesh,
        scratch_types=[
            pltpu.VMEM((L, N), jnp.float32),
            pltpu.VMEM((Q, 2 * N), jnp.float32),
            pltpu.VMEM((Q, 2 * N), jnp.float32),
            pltpu.SemaphoreType.DMA,
            pltpu.SemaphoreType.DMA,
        ],
    )
    def run(inc_hbm, x_hbm, table_v, slab0, slab1, sem0, sem1):
        wid = lax.axis_index("s") * NC + lax.axis_index("c")
        tpb = L // slabs_per_tile
        b = wid // tpb
        r0 = (wid % tpb) * slabs_per_tile

        pltpu.sync_copy(inc_hbm.at[b], table_v)

        slabs = (slab0, slab1)
        sems = (sem0, sem1)

        def build(slab, r):
            s = lax.shift_right_logical(r, 1)
            d = lax.bitwise_and(r, 1)

            def q_body(q, carry):
                m0 = s + q
                m1 = m0 + d
                for k in range(KN):
                    slab[q, pl.ds(k * _LANES, _LANES)] = (
                        table_v[m0, pl.ds(k * _LANES, _LANES)]
                    )
                    slab[q, pl.ds(N + k * _LANES, _LANES)] = (
                        table_v[m1, pl.ds(k * _LANES, _LANES)]
                    )
                return carry

            lax.fori_loop(0, Q, q_body, 0)

        def outer(g, carry):
            for buf in range(2):
                r = r0 + g * 2 + buf

                @pl.when(g > 0)
                def _wait():
                    pltpu.make_async_copy(
                        slabs[buf], x_hbm.at[b, r0], sems[buf]
                    ).wait()

                build(slabs[buf], r)
                pltpu.async_copy(slabs[buf], x_hbm.at[b, r], sems[buf])
            return carry

        lax.fori_loop(0, slabs_per_tile // 2, outer, 0)

        for buf in range(2):
            pltpu.make_async_copy(
                slabs[buf], x_hbm.at[b, r0], sems[buf]
            ).wait()

    return run


@functools.lru_cache(maxsize=None)
def _make_full_sc(B, L, N, out_B=None):
    if out_B is None:
        out_B = B
    info = plsc.get_sparse_core_info()
    NC, NS = info.num_cores, info.num_subcores
    NW = NC * NS
    assert (B * L) % NW == 0
    slabs_per_tile = (B * L) // NW
    KN = N // _LANES

    mesh = plsc.VectorSubcoreMesh(core_axis_name="c", subcore_axis_name="s")

    @functools.partial(
        pl.kernel,
        out_type=jax.ShapeDtypeStruct((out_B, L, L, 3 * N), jnp.float32),
        mesh=mesh,
        scratch_types=[
            pltpu.VMEM((L, N), jnp.float32),
            pltpu.VMEM((L, 3 * N), jnp.float32),
            pltpu.VMEM((L, 3 * N), jnp.float32),
            pltpu.SemaphoreType.DMA,
            pltpu.SemaphoreType.DMA,
        ],
    )
    def run(inc_hbm, out_hbm, table_v, slab0, slab1, sem0, sem1):
        wid = lax.axis_index("s") * NC + lax.axis_index("c")
        tpb = L // slabs_per_tile
        b = wid // tpb
        r0 = (wid % tpb) * slabs_per_tile

        pltpu.sync_copy(inc_hbm.at[b], table_v)

        slabs = (slab0, slab1)
        sems = (sem0, sem1)

        def init_body(c, carry):
            for k in range(KN):
                v = table_v[c, pl.ds(k * _LANES, _LANES)]
                slab0[c, pl.ds(k * _LANES, _LANES)] = v
                slab1[c, pl.ds(k * _LANES, _LANES)] = v
            return carry

        lax.fori_loop(0, L, init_body, 0)

        def build(slab, r):
            row = [table_v[r, pl.ds(k * _LANES, _LANES)] for k in range(KN)]

            def c_body(c, carry):
                m = lax.shift_right_logical(r + c, 1)
                for k in range(KN):
                    slab[c, pl.ds(N + k * _LANES, _LANES)] = row[k]
                    slab[c, pl.ds(2 * N + k * _LANES, _LANES)] = (
                        table_v[m, pl.ds(k * _LANES, _LANES)]
                    )
                return carry

            lax.fori_loop(0, L, c_body, 0)

        def outer(g, carry):
            for buf in range(2):
                r = r0 + g * 2 + buf

                @pl.when(g > 0)
                def _wait():
                    pltpu.make_async_copy(
                        slabs[buf], out_hbm.at[b, r0], sems[buf]
                    ).wait()

                build(slabs[buf], r)
                pltpu.async_copy(slabs[buf], out_hbm.at[b, r], sems[buf])
            return carry

        lax.fori_loop(0, slabs_per_tile // 2, outer, 0)

        for buf in range(2):
            pltpu.make_async_copy(
                slabs[buf], out_hbm.at[b, r0], sems[buf]
            ).wait()

    return run


@functools.lru_cache(maxsize=None)
def _make_tc_assemble(B, L, N):
    def body(inc_ref, x_ref, o_ref):
        r = pl.program_id(1)
        t = inc_ref[0]
        sel = lax.broadcasted_iota(jnp.int32, (L, L), 1)
        s2 = (sel == r).astype(jnp.float32)
        p2 = jnp.dot(s2, t, preferred_element_type=jnp.float32)
        p3 = x_ref[0, 0].reshape(L, N)
        o_ref[0, 0] = jnp.concatenate([t, p2, p3], axis=1)

    return pl.pallas_call(
        body,
        grid=(B, L),
        in_specs=[
            pl.BlockSpec((1, L, N), lambda b, r: (b, 0, 0)),
            pl.BlockSpec((1, 1, L // 2, 2 * N), lambda b, r: (b, r, 0, 0)),
        ],
        out_specs=pl.BlockSpec((1, 1, L, 3 * N), lambda b, r: (b, r, 0, 0)),
        out_shape=jax.ShapeDtypeStruct((B, L, L, 3 * N), jnp.float32),
    )


def kernel(incoming):
    B, L, N = incoming.shape
    z = jnp.zeros((B, L, N), jnp.float32)
    a1 = jnp.concatenate([incoming, z, z], axis=-1)
    a2 = jnp.concatenate([z, incoming, z], axis=-1)
    out12 = a1[:, None, :, :] + a2[:, :, None, :]
    x = _make_sc_gather(B, L, N)(incoming)
    p3 = x.reshape(B, L, L, N)
    return lax.dynamic_update_slice(out12, p3, (0, 0, 0, 2 * N))


# speedup vs baseline: 1.1186x; 1.1178x over previous
import functools

import jax
import jax.numpy as jnp
from jax import lax
from jax.experimental import pallas as pl
from jax.experimental.pallas import tpu as pltpu
from jax.experimental.pallas import tpu_sc as plsc

_LANES = 16


@functools.lru_cache(maxsize=None)
def _make_full_sc(B, L, N):
    info = plsc.get_sparse_core_info()
    NC, NS = info.num_cores, info.num_subcores
    NW = NC * NS
    assert (B * L) % NW == 0
    slabs_per_tile = (B * L) // NW
    assert L % slabs_per_tile == 0 and slabs_per_tile % 2 == 0
    KN = N // _LANES

    mesh = plsc.VectorSubcoreMesh(core_axis_name="c", subcore_axis_name="s")

    @functools.partial(
        pl.kernel,
        out_type=jax.ShapeDtypeStruct((B, L, L, 3 * N), jnp.float32),
        mesh=mesh,
        scratch_types=[
            pltpu.VMEM((L, N), jnp.float32),
            pltpu.VMEM((L, 3 * N), jnp.float32),
            pltpu.VMEM((L, 3 * N), jnp.float32),
            pltpu.SemaphoreType.DMA,
            pltpu.SemaphoreType.DMA,
        ],
    )
    def run(inc_hbm, out_hbm, table_v, slab0, slab1, sem0, sem1):
        wid = lax.axis_index("s") * NC + lax.axis_index("c")
        tpb = L // slabs_per_tile
        b = wid // tpb
        r0 = (wid % tpb) * slabs_per_tile

        pltpu.sync_copy(inc_hbm.at[b], table_v)

        slabs = (slab0, slab1)
        sems = (sem0, sem1)

        def init_body(c, carry):
            for k in range(KN):
                v = table_v[c, pl.ds(k * _LANES, _LANES)]
                slab0[c, pl.ds(k * _LANES, _LANES)] = v
                slab1[c, pl.ds(k * _LANES, _LANES)] = v
            return carry

        lax.fori_loop(0, L, init_body, 0)

        def build(slab, r):
            row = [table_v[r, pl.ds(k * _LANES, _LANES)] for k in range(KN)]

            def c_body(c, carry):
                m = lax.shift_right_logical(r + c, 1)
                for k in range(KN):
                    slab[c, pl.ds(N + k * _LANES, _LANES)] = row[k]
                    slab[c, pl.ds(2 * N + k * _LANES, _LANES)] = (
                        table_v[m, pl.ds(k * _LANES, _LANES)]
                    )
                return carry

            lax.fori_loop(0, L, c_body, 0)

        def outer(g, carry):
            for buf in range(2):
                r = r0 + g * 2 + buf

                @pl.when(g > 0)
                def _wait():
                    pltpu.make_async_copy(
                        slabs[buf], out_hbm.at[b, r0], sems[buf]
                    ).wait()

                build(slabs[buf], r)
                pltpu.async_copy(slabs[buf], out_hbm.at[b, r], sems[buf])
            return carry

        lax.fori_loop(0, slabs_per_tile // 2, outer, 0)

        for buf in range(2):
            pltpu.make_async_copy(
                slabs[buf], out_hbm.at[b, r0], sems[buf]
            ).wait()

    return run


def kernel(incoming):
    B, L, N = incoming.shape
    out = _make_full_sc(B, L, N)(incoming)
    tail = lax.dynamic_slice(out, (B - 1, L - 1, 0, 0), (1, 1, L, 3 * N))
    return lax.dynamic_update_slice(out, tail, (B - 1, L - 1, 0, 0))
